# baseline (speedup 1.0000x reference)
import functools

import jax
import jax.numpy as jnp
from jax import lax
from jax.experimental import pallas as pl
from jax.experimental.pallas import tpu as pltpu
from jax.experimental.pallas import tpu_sc as plsc

NROW = 1_000_000
NCOL = 64
NPAIR = 1_048_576
NC = 2
NS = 16
NW = NC * NS
L = 16
CW = 6912
NJ = 145
CWLAST = 4736
NBAND = NCOL // 8
NB = NBAND * NJ
NBP = 1280
PPW = NPAIR // NW
WIN = 2048
CAP = ((PPW + NBP * 7 + 7) // 8) * 8
SEG = 128
PART_LEN = NW * CAP + SEG
META_LEN = NBP * NW


def _scan_base():
    cnt, _ = plsc.scan_count(jnp.zeros((L,), jnp.int32))
    return cnt[0]


def _remap(ivec):
    i32 = jnp.int32
    c = jnp.bitwise_and(ivec, i32(63))
    r = lax.shift_right_logical(ivec, 6)
    jj = lax.div(r, i32(CW))
    b = lax.shift_right_logical(c, 3) * NJ + jj
    off = jnp.bitwise_or(lax.shift_left(jnp.bitwise_and(c, i32(7)), 13),
                         r - jj * CW)
    return b, jnp.bitwise_or(lax.shift_left(b, 16), off)


def _phase_a(idx_hbm, src_hbm, ipart_hbm, spart_hbm, meta_hbm,
             iwin, swin, li, ls, hist, cur, mpack, ibatch, sem, semA, semB):
    cid = lax.axis_index("c")
    sid = lax.axis_index("s")
    wid = cid * NS + sid
    pbase = pl.multiple_of(wid * PPW, 8)
    i32 = jnp.int32
    iota = lax.broadcasted_iota(i32, (L,), 0)
    zero16 = jnp.zeros((L,), i32)
    zf = jnp.zeros((L,), jnp.float32)
    sbase = _scan_base()

    def zero_hist(k, carry):
        hist[pl.ds(k * L, L)] = zero16
        return carry

    lax.fori_loop(0, NBP // L, zero_hist, i32(0))

    pltpu.async_copy(idx_hbm.at[pl.ds(pbase, WIN)],
                     iwin.at[pl.ds(0, WIN)], semA)

    def hist_win(w, carry):
        par = jnp.bitwise_and(w, i32(1))
        half = par * WIN

        @pl.when(par == 0)
        def _():
            pltpu.make_async_copy(idx_hbm.at[pl.ds(0, WIN)],
                                  iwin.at[pl.ds(0, WIN)], semA).wait()

        @pl.when(par == 1)
        def _():
            pltpu.make_async_copy(idx_hbm.at[pl.ds(0, WIN)],
                                  iwin.at[pl.ds(0, WIN)], semB).wait()

        @pl.when(jnp.logical_and(par == 0, w < PPW // WIN - 1))
        def _():
            pltpu.async_copy(idx_hbm.at[pl.ds(pbase + (w + 1) * WIN, WIN)],
                             iwin.at[pl.ds(WIN, WIN)], semB)

        @pl.when(jnp.logical_and(par == 1, w < PPW // WIN - 1))
        def _():
            pltpu.async_copy(idx_hbm.at[pl.ds(pbase + (w + 1) * WIN, WIN)],
                             iwin.at[pl.ds(0, WIN)], semA)

        def hist_vreg(v, c2):
            b, _ = _remap(iwin[pl.ds(half + v * L, L)])
            r, islast = plsc.scan_count(b)
            tot = r - sbase + 1
            plsc.addupdate_scatter(hist, [b], tot, mask=islast)
            return c2

        return lax.fori_loop(0, WIN // L, hist_vreg, carry)

    lax.fori_loop(0, PPW // WIN, hist_win, i32(0))

    def scan_bins(k, carry):
        v = hist[pl.ds(k * L, L)]
        cpad = jnp.bitwise_and(v + 7, i32(-8))
        inc = plsc.cumsum(cpad)
        loc = carry + inc - cpad
        cur[pl.ds(k * L, L)] = loc
        mpack[pl.ds(k * L, L)] = jnp.bitwise_or(
            loc, lax.shift_left(cpad, 16))
        return carry + inc[L - 1]

    lax.fori_loop(0, NBP // L, scan_bins, i32(0))

    for g in range(NBP // SEG):
        for j in range(SEG // L):
            r = g * SEG + j * L + iota
            ibatch[g, pl.ds(j * L, L)] = r * NW + wid
    for g in range(NBP // SEG):
        pltpu.sync_copy(mpack.at[pl.ds(g * SEG, SEG)],
                        meta_hbm.at[ibatch.at[g]])

    def pad_fill(k, carry):
        cnt16 = hist[pl.ds(k * L, L)]
        cpad16 = jnp.bitwise_and(cnt16 + 7, i32(-8))
        lo16 = cur[pl.ds(k * L, L)]
        bin16 = k * L + iota
        for j in range(7):
            mask = (cnt16 + j) < cpad16
            dest = lo16 + cnt16 + j
            plsc.store_scatter(
                li, [dest], lax.shift_left(bin16, 16) + j * 64, mask=mask)
            plsc.store_scatter(ls, [dest], zf, mask=mask)
        return carry

    lax.fori_loop(0, NBP // L, pad_fill, i32(0))

    pltpu.async_copy(idx_hbm.at[pl.ds(pbase, WIN)],
                     iwin.at[pl.ds(0, WIN)], semA)
    pltpu.async_copy(src_hbm.at[pl.ds(pbase, WIN)],
                     swin.at[pl.ds(0, WIN)], semA)

    def scat_win(w, carry):
        par = jnp.bitwise_and(w, i32(1))
        half = par * WIN

        @pl.when(par == 0)
        def _():
            pltpu.make_async_copy(idx_hbm.at[pl.ds(0, WIN)],
                                  iwin.at[pl.ds(0, WIN)], semA).wait()
            pltpu.make_async_copy(src_hbm.at[pl.ds(0, WIN)],
                                  swin.at[pl.ds(0, WIN)], semA).wait()

        @pl.when(par == 1)
        def _():
            pltpu.make_async_copy(idx_hbm.at[pl.ds(0, WIN)],
                                  iwin.at[pl.ds(0, WIN)], semB).wait()
            pltpu.make_async_copy(src_hbm.at[pl.ds(0, WIN)],
                                  swin.at[pl.ds(0, WIN)], semB).wait()

        @pl.when(jnp.logical_and(par == 0, w < PPW // WIN - 1))
        def _():
            pltpu.async_copy(idx_hbm.at[pl.ds(pbase + (w + 1) * WIN, WIN)],
                             iwin.at[pl.ds(WIN, WIN)], semB)
            pltpu.async_copy(src_hbm.at[pl.ds(pbase + (w + 1) * WIN, WIN)],
                             swin.at[pl.ds(WIN, WIN)], semB)

        @pl.when(jnp.logical_and(par == 1, w < PPW // WIN - 1))
        def _():
            pltpu.async_copy(idx_hbm.at[pl.ds(pbase + (w + 1) * WIN, WIN)],
                             iwin.at[pl.ds(0, WIN)], semA)
            pltpu.async_copy(src_hbm.at[pl.ds(pbase + (w + 1) * WIN, WIN)],
                             swin.at[pl.ds(0, WIN)], semA)

        def scat_vreg(v, c2):
            svec = swin[pl.ds(half + v * L, L)]
            b, iq = _remap(iwin[pl.ds(half + v * L, L)])
            r, islast = plsc.scan_count(b)
            rex = r - sbase
            curv = plsc.load_gather(cur, [b])
            dest = curv + rex
            plsc.store_scatter(li, [dest], iq)
            plsc.store_scatter(ls, [dest], svec)
            plsc.addupdate_scatter(cur, [b], rex + 1, mask=islast)
            return c2

        return lax.fori_loop(0, WIN // L, scat_vreg, carry)

    lax.fori_loop(0, PPW // WIN, scat_win, i32(0))

    rbase = pl.multiple_of(wid * CAP, 8)
    pltpu.sync_copy(li, ipart_hbm.at[pl.ds(rbase, CAP)])
    pltpu.sync_copy(ls, spart_hbm.at[pl.ds(rbase, CAP)])


def _phase_b(xt_hbm, ipart_hbm, spart_hbm, meta_hbm, out_hbm,
             cb0, cb1, ibuf0, sbuf0, mrow0, ibuf1, sbuf1, mrow1,
             semx0, semo0, semx1, semo1, semp):
    cid = lax.axis_index("c")
    sid = lax.axis_index("s")
    wid = cid * NS + sid
    i32 = jnp.int32
    iota = lax.broadcasted_iota(i32, (L,), 0)

    def extract(ref, t):
        return jnp.max(plsc.load_gather(ref, [jnp.full((L,), t, i32)]))

    def split_c(cc):
        cv = jnp.full((L,), cc, i32)
        bandv = lax.div(cv, i32(NJ))
        band = jnp.max(bandv)
        jj = jnp.max(cv - bandv * NJ)
        return band, jj

    def read_mrow(cc, mrow):
        pltpu.sync_copy(meta_hbm.at[pl.ds(pl.multiple_of(cc * NW, 8), NW)],
                        mrow)

    def fire_seg(t, iset):
        ibuf, sbuf, mrow = iset
        packed = extract(mrow, t)
        lo = jnp.bitwise_and(packed, i32(0xFFFF))
        segbase = pl.multiple_of(t * CAP + lo, 8)
        pltpu.async_copy(ipart_hbm.at[pl.ds(segbase, SEG)], ibuf.at[t], semp)
        pltpu.async_copy(spart_hbm.at[pl.ds(segbase, SEG)], sbuf.at[t], semp)

    def drain_seg(t, iset):
        ibuf, sbuf, _ = iset
        pltpu.make_async_copy(
            ipart_hbm.at[pl.ds(0, SEG)], ibuf.at[t], semp).wait()
        pltpu.make_async_copy(
            spart_hbm.at[pl.ds(0, SEG)], sbuf.at[t], semp).wait()

    def drain_out(buf, sem, cc):
        _, pjj = split_c(cc)

        @pl.when(pjj < NJ - 1)
        def _():
            pltpu.make_async_copy(
                xt_hbm.at[pl.ds(0, 8), pl.ds(0, CW)], buf, sem).wait()

        @pl.when(pjj == NJ - 1)
        def _():
            pltpu.make_async_copy(
                xt_hbm.at[pl.ds(0, 8), pl.ds(0, CWLAST)],
                buf.at[:, pl.ds(0, CWLAST)], sem).wait()

    read_mrow(wid, mrow0)

    def prime(t, carry):
        fire_seg(t, (ibuf0, sbuf0, mrow0))
        return carry

    lax.fori_loop(0, NW, prime, i32(0))

    def sub_step(k, buf, semx, semo, cur_set, next_set):
        c = k * NW + wid

        @pl.when(c < NB)
        def _body():
            band, jj = split_c(c)
            row0 = pl.multiple_of(band * 8, 8)
            col0 = pl.multiple_of(jj * CW, 128)

            @pl.when(k >= 2)
            def _():
                drain_out(buf, semo, c - 2 * NW)

            @pl.when(jj < NJ - 1)
            def _():
                pltpu.async_copy(
                    xt_hbm.at[pl.ds(row0, 8), pl.ds(col0, CW)], buf, semx)

            @pl.when(jj == NJ - 1)
            def _():
                pltpu.async_copy(
                    xt_hbm.at[pl.ds(row0, 8), pl.ds(col0, CWLAST)],
                    buf.at[:, pl.ds(0, CWLAST)], semx)

            nc = (k + 1) * NW + wid

            @pl.when(nc < NB)
            def _():
                read_mrow(nc, next_set[2])

                def fd(t, c2):
                    fire_seg(t, next_set)
                    drain_seg(t, cur_set)
                    return c2

                lax.fori_loop(0, NW, fd, i32(0))

            @pl.when(nc >= NB)
            def _():
                def donly(t, c2):
                    drain_seg(t, cur_set)
                    return c2

                lax.fori_loop(0, NW, donly, i32(0))

            @pl.when(jj < NJ - 1)
            def _():
                pltpu.make_async_copy(
                    xt_hbm.at[pl.ds(0, 8), pl.ds(0, CW)], buf, semx).wait()

            @pl.when(jj == NJ - 1)
            def _():
                pltpu.make_async_copy(
                    xt_hbm.at[pl.ds(0, 8), pl.ds(0, CWLAST)],
                    buf.at[:, pl.ds(0, CWLAST)], semx).wait()

            ibuf, sbuf, mrow = cur_set

            def apply_region(t, c2):
                packed = extract(mrow, t)
                lo = jnp.bitwise_and(packed, i32(0xFFFF))
                cnt = jnp.bitwise_and(
                    lax.shift_right_logical(packed, 16), i32(0xFFFF))

                def apply_vreg(v, c3):
                    iq = ibuf[t, pl.ds(v * L, L)]
                    svec = sbuf[t, pl.ds(v * L, L)]
                    mask = (v * L + iota) < cnt
                    row = jnp.bitwise_and(
                        lax.shift_right_logical(iq, 13), i32(7))
                    col = jnp.bitwise_and(iq, i32(8191))
                    plsc.addupdate_scatter(buf, [row, col], svec, mask=mask)
                    return c3

                nv = lax.div(jnp.minimum(cnt, SEG) + (L - 1), i32(L))
                lax.fori_loop(0, nv, apply_vreg, i32(0))

                segbase = pl.multiple_of(t * CAP + lo, 8)

                def cont(pos):
                    wb = pl.multiple_of(segbase + pos, 8)
                    pltpu.sync_copy(ipart_hbm.at[pl.ds(wb, SEG)], ibuf.at[t])
                    pltpu.sync_copy(spart_hbm.at[pl.ds(wb, SEG)], sbuf.at[t])

                    def cont_vreg(v, c4):
                        iq = ibuf[t, pl.ds(v * L, L)]
                        svec = sbuf[t, pl.ds(v * L, L)]
                        mask = (pos + v * L + iota) < cnt
                        row = jnp.bitwise_and(
                            lax.shift_right_logical(iq, 13), i32(7))
                        col = jnp.bitwise_and(iq, i32(8191))
                        plsc.addupdate_scatter(
                            buf, [row, col], svec, mask=mask)
                        return c4

                    lax.fori_loop(0, SEG // L, cont_vreg, i32(0))
                    return pos + SEG

                lax.while_loop(lambda pos: pos < cnt, cont, i32(SEG))
                return c2

            lax.fori_loop(0, NW, apply_region, i32(0))

            @pl.when(jj < NJ - 1)
            def _():
                pltpu.async_copy(
                    buf, out_hbm.at[pl.ds(row0, 8), pl.ds(col0, CW)], semo)

            @pl.when(jj == NJ - 1)
            def _():
                pltpu.async_copy(
                    buf.at[:, pl.ds(0, CWLAST)],
                    out_hbm.at[pl.ds(row0, 8), pl.ds(col0, CWLAST)], semo)

    set0 = (ibuf0, sbuf0, mrow0)
    set1 = (ibuf1, sbuf1, mrow1)

    def chunk_loop(k2, carry):
        sub_step(2 * k2, cb0, semx0, semo0, set0, set1)
        sub_step(2 * k2 + 1, cb1, semx1, semo1, set1, set0)
        return carry

    lax.fori_loop(0, NBP // NW // 2, chunk_loop, i32(0))

    lastk = lax.div(i32(NB - 1) - wid, i32(NW))
    lastc = lastk * NW + wid
    parity = jnp.bitwise_and(lastk, i32(1))

    @pl.when(parity == 0)
    def _():
        drain_out(cb0, semo0, lastc)
        drain_out(cb1, semo1, lastc - NW)

    @pl.when(parity == 1)
    def _():
        drain_out(cb1, semo1, lastc)
        drain_out(cb0, semo0, lastc - NW)


def kernel(x, index, source):
    i32 = jnp.int32
    f32 = jnp.float32
    xt = x.T
    idx = index.astype(i32)
    mesh = plsc.VectorSubcoreMesh(core_axis_name="c", subcore_axis_name="s")
    params_a = pltpu.CompilerParams(needs_layout_passes=False)
    params_b = pltpu.CompilerParams(needs_layout_passes=False,
                                    use_tc_tiling_on_sc=True)

    phase_a = functools.partial(
        pl.kernel, mesh=mesh, compiler_params=params_a,
        out_type=[jax.ShapeDtypeStruct((PART_LEN,), i32),
                  jax.ShapeDtypeStruct((PART_LEN,), f32),
                  jax.ShapeDtypeStruct((META_LEN,), i32)],
        scratch_types=[
            pltpu.VMEM((2 * WIN,), i32),
            pltpu.VMEM((2 * WIN,), f32),
            pltpu.VMEM((CAP,), i32),
            pltpu.VMEM((CAP,), f32),
            pltpu.VMEM((NBP,), i32),
            pltpu.VMEM((NBP,), i32),
            pltpu.VMEM((NBP,), i32),
            pltpu.VMEM((NBP // SEG, SEG), i32),
            pltpu.SemaphoreType.DMA,
            pltpu.SemaphoreType.DMA,
            pltpu.SemaphoreType.DMA,
        ])(_phase_a)
    ipart, spart, meta = phase_a(idx, source)

    phase_b = functools.partial(
        pl.kernel, mesh=mesh, compiler_params=params_b,
        out_type=jax.ShapeDtypeStruct((NCOL, NROW), f32),
        scratch_types=[
            pltpu.VMEM((8, CW), f32),
            pltpu.VMEM((8, CW), f32),
            pltpu.VMEM((NW, SEG), i32),
            pltpu.VMEM((NW, SEG), f32),
            pltpu.VMEM((NW,), i32),
            pltpu.VMEM((NW, SEG), i32),
            pltpu.VMEM((NW, SEG), f32),
            pltpu.VMEM((NW,), i32),
            pltpu.SemaphoreType.DMA,
            pltpu.SemaphoreType.DMA,
            pltpu.SemaphoreType.DMA,
            pltpu.SemaphoreType.DMA,
            pltpu.SemaphoreType.DMA,
        ])(_phase_b)
    out = phase_b(xt, ipart, spart, meta)
    return out.T


# speedup vs baseline: 1.0319x; 1.0319x over previous
import functools

import jax
import jax.numpy as jnp
from jax import lax
from jax.experimental import pallas as pl
from jax.experimental.pallas import tpu as pltpu
from jax.experimental.pallas import tpu_sc as plsc

NROW = 1_000_000
NCOL = 64
NPAIR = 1_048_576
NC = 2
NS = 16
NW = NC * NS
L = 16
CW = 6912
NJ = 145
CWLAST = 4736
NBAND = NCOL // 8
NB = NBAND * NJ
NBP = 1280
PPW = NPAIR // NW
WIN = 2048
CAP = ((PPW + NBP * 7 + 7) // 8) * 8
SEG = 128
PART_LEN = NW * CAP + SEG
META_LEN = NBP * NW


def _scan_base():
    cnt, _ = plsc.scan_count(jnp.zeros((L,), jnp.int32))
    return cnt[0]


def _remap(ivec):
    i32 = jnp.int32
    c = jnp.bitwise_and(ivec, i32(63))
    r = lax.shift_right_logical(ivec, 6)
    jj = lax.div(r, i32(CW))
    b = lax.shift_right_logical(c, 3) * NJ + jj
    off = jnp.bitwise_or(lax.shift_left(jnp.bitwise_and(c, i32(7)), 13),
                         r - jj * CW)
    return b, jnp.bitwise_or(lax.shift_left(b, 16), off)


def _phase_a(idx_hbm, src_hbm, ipart_hbm, spart_hbm, meta_hbm,
             iwin, swin, li, ls, hist, cur, mpack, ibatch, sem):
    cid = lax.axis_index("c")
    sid = lax.axis_index("s")
    wid = cid * NS + sid
    pbase = pl.multiple_of(wid * PPW, 8)
    i32 = jnp.int32
    iota = lax.broadcasted_iota(i32, (L,), 0)
    zero16 = jnp.zeros((L,), i32)
    zf = jnp.zeros((L,), jnp.float32)
    sbase = _scan_base()

    def zero_hist(k, carry):
        hist[pl.ds(k * L, L)] = zero16
        return carry

    lax.fori_loop(0, NBP // L, zero_hist, i32(0))

    def hist_win(w, carry):
        pltpu.sync_copy(idx_hbm.at[pl.ds(pbase + w * WIN, WIN)], iwin)

        def hist_vreg(v, c2):
            b, _ = _remap(iwin[pl.ds(v * L, L)])
            r, islast = plsc.scan_count(b)
            tot = r - sbase + 1
            plsc.addupdate_scatter(hist, [b], tot, mask=islast)
            return c2

        return lax.fori_loop(0, WIN // L, hist_vreg, carry)

    lax.fori_loop(0, PPW // WIN, hist_win, i32(0))

    def scan_bins(k, carry):
        v = hist[pl.ds(k * L, L)]
        cpad = jnp.bitwise_and(v + 7, i32(-8))
        inc = plsc.cumsum(cpad)
        loc = carry + inc - cpad
        cur[pl.ds(k * L, L)] = loc
        mpack[pl.ds(k * L, L)] = jnp.bitwise_or(
            loc, lax.shift_left(cpad, 16))
        return carry + inc[L - 1]

    lax.fori_loop(0, NBP // L, scan_bins, i32(0))

    for g in range(NBP // SEG):
        for j in range(SEG // L):
            r = g * SEG + j * L + iota
            ibatch[g, pl.ds(j * L, L)] = r * NW + wid
    for g in range(NBP // SEG):
        pltpu.sync_copy(mpack.at[pl.ds(g * SEG, SEG)],
                        meta_hbm.at[ibatch.at[g]])

    def pad_fill(k, carry):
        cnt16 = hist[pl.ds(k * L, L)]
        cpad16 = jnp.bitwise_and(cnt16 + 7, i32(-8))
        lo16 = cur[pl.ds(k * L, L)]
        bin16 = k * L + iota
        for j in range(7):
            mask = (cnt16 + j) < cpad16
            dest = lo16 + cnt16 + j
            plsc.store_scatter(
                li, [dest], lax.shift_left(bin16, 16) + j * 64, mask=mask)
            plsc.store_scatter(ls, [dest], zf, mask=mask)
        return carry

    lax.fori_loop(0, NBP // L, pad_fill, i32(0))

    def scat_win(w, carry):
        pltpu.sync_copy(idx_hbm.at[pl.ds(pbase + w * WIN, WIN)], iwin)
        pltpu.sync_copy(src_hbm.at[pl.ds(pbase + w * WIN, WIN)], swin)

        def scat_vreg(v, c2):
            svec = swin[pl.ds(v * L, L)]
            b, iq = _remap(iwin[pl.ds(v * L, L)])
            r, islast = plsc.scan_count(b)
            rex = r - sbase
            curv = plsc.load_gather(cur, [b])
            dest = curv + rex
            plsc.store_scatter(li, [dest], iq)
            plsc.store_scatter(ls, [dest], svec)
            plsc.addupdate_scatter(cur, [b], rex + 1, mask=islast)
            return c2

        return lax.fori_loop(0, WIN // L, scat_vreg, carry)

    lax.fori_loop(0, PPW // WIN, scat_win, i32(0))

    rbase = pl.multiple_of(wid * CAP, 8)
    pltpu.sync_copy(li, ipart_hbm.at[pl.ds(rbase, CAP)])
    pltpu.sync_copy(ls, spart_hbm.at[pl.ds(rbase, CAP)])


def _phase_b(xt_hbm, ipart_hbm, spart_hbm, meta_hbm, out_hbm,
             cb0, cb1, ibuf0, sbuf0, mrow0, ibuf1, sbuf1, mrow1,
             semx0, semo0, semx1, semo1, semp):
    cid = lax.axis_index("c")
    sid = lax.axis_index("s")
    wid = cid * NS + sid
    i32 = jnp.int32
    iota = lax.broadcasted_iota(i32, (L,), 0)

    def extract(ref, t):
        return jnp.max(plsc.load_gather(ref, [jnp.full((L,), t, i32)]))

    def split_c(cc):
        cv = jnp.full((L,), cc, i32)
        bandv = lax.div(cv, i32(NJ))
        band = jnp.max(bandv)
        jj = jnp.max(cv - bandv * NJ)
        return band, jj

    def read_mrow(cc, mrow):
        pltpu.sync_copy(meta_hbm.at[pl.ds(pl.multiple_of(cc * NW, 8), NW)],
                        mrow)

    def fire_seg(t, iset):
        ibuf, sbuf, mrow = iset
        packed = extract(mrow, t)
        lo = jnp.bitwise_and(packed, i32(0xFFFF))
        segbase = pl.multiple_of(t * CAP + lo, 8)
        pltpu.async_copy(ipart_hbm.at[pl.ds(segbase, SEG)], ibuf.at[t], semp)
        pltpu.async_copy(spart_hbm.at[pl.ds(segbase, SEG)], sbuf.at[t], semp)

    def drain_seg(t, iset):
        ibuf, sbuf, _ = iset
        pltpu.make_async_copy(
            ipart_hbm.at[pl.ds(0, SEG)], ibuf.at[t], semp).wait()
        pltpu.make_async_copy(
            spart_hbm.at[pl.ds(0, SEG)], sbuf.at[t], semp).wait()

    def drain_out(buf, sem, cc):
        _, pjj = split_c(cc)

        @pl.when(pjj < NJ - 1)
        def _():
            pltpu.make_async_copy(
                xt_hbm.at[pl.ds(0, 8), pl.ds(0, CW)], buf, sem).wait()

        @pl.when(pjj == NJ - 1)
        def _():
            pltpu.make_async_copy(
                xt_hbm.at[pl.ds(0, 8), pl.ds(0, CWLAST)],
                buf.at[:, pl.ds(0, CWLAST)], sem).wait()

    read_mrow(wid, mrow0)

    def prime(t, carry):
        fire_seg(t, (ibuf0, sbuf0, mrow0))
        return carry

    lax.fori_loop(0, NW, prime, i32(0))

    def sub_step(k, buf, semx, semo, cur_set, next_set):
        c = k * NW + wid

        @pl.when(c < NB)
        def _body():
            band, jj = split_c(c)
            row0 = pl.multiple_of(band * 8, 8)
            col0 = pl.multiple_of(jj * CW, 128)

            @pl.when(k >= 2)
            def _():
                drain_out(buf, semo, c - 2 * NW)

            @pl.when(jj < NJ - 1)
            def _():
                pltpu.async_copy(
                    xt_hbm.at[pl.ds(row0, 8), pl.ds(col0, CW)], buf, semx)

            @pl.when(jj == NJ - 1)
            def _():
                pltpu.async_copy(
                    xt_hbm.at[pl.ds(row0, 8), pl.ds(col0, CWLAST)],
                    buf.at[:, pl.ds(0, CWLAST)], semx)

            nc = (k + 1) * NW + wid

            @pl.when(nc < NB)
            def _():
                read_mrow(nc, next_set[2])

                def fd(t, c2):
                    fire_seg(t, next_set)
                    drain_seg(t, cur_set)
                    return c2

                lax.fori_loop(0, NW, fd, i32(0))

            @pl.when(nc >= NB)
            def _():
                def donly(t, c2):
                    drain_seg(t, cur_set)
                    return c2

                lax.fori_loop(0, NW, donly, i32(0))

            @pl.when(jj < NJ - 1)
            def _():
                pltpu.make_async_copy(
                    xt_hbm.at[pl.ds(0, 8), pl.ds(0, CW)], buf, semx).wait()

            @pl.when(jj == NJ - 1)
            def _():
                pltpu.make_async_copy(
                    xt_hbm.at[pl.ds(0, 8), pl.ds(0, CWLAST)],
                    buf.at[:, pl.ds(0, CWLAST)], semx).wait()

            ibuf, sbuf, mrow = cur_set

            def apply_region(t, c2):
                packed = extract(mrow, t)
                lo = jnp.bitwise_and(packed, i32(0xFFFF))
                cnt = jnp.bitwise_and(
                    lax.shift_right_logical(packed, 16), i32(0xFFFF))

                def apply_vreg(v, c3):
                    iq = ibuf[t, pl.ds(v * L, L)]
                    svec = sbuf[t, pl.ds(v * L, L)]
                    mask = (v * L + iota) < cnt
                    row = jnp.bitwise_and(
                        lax.shift_right_logical(iq, 13), i32(7))
                    col = jnp.bitwise_and(iq, i32(8191))
                    plsc.addupdate_scatter(buf, [row, col], svec, mask=mask)
                    return c3

                nv = lax.div(jnp.minimum(cnt, SEG) + (L - 1), i32(L))
                lax.fori_loop(0, nv, apply_vreg, i32(0))

                segbase = pl.multiple_of(t * CAP + lo, 8)

                def cont(pos):
                    wb = pl.multiple_of(segbase + pos, 8)
                    pltpu.sync_copy(ipart_hbm.at[pl.ds(wb, SEG)], ibuf.at[t])
                    pltpu.sync_copy(spart_hbm.at[pl.ds(wb, SEG)], sbuf.at[t])

                    def cont_vreg(v, c4):
                        iq = ibuf[t, pl.ds(v * L, L)]
                        svec = sbuf[t, pl.ds(v * L, L)]
                        mask = (pos + v * L + iota) < cnt
                        row = jnp.bitwise_and(
                            lax.shift_right_logical(iq, 13), i32(7))
                        col = jnp.bitwise_and(iq, i32(8191))
                        plsc.addupdate_scatter(
                            buf, [row, col], svec, mask=mask)
                        return c4

                    lax.fori_loop(0, SEG // L, cont_vreg, i32(0))
                    return pos + SEG

                lax.while_loop(lambda pos: pos < cnt, cont, i32(SEG))
                return c2

            lax.fori_loop(0, NW, apply_region, i32(0))

            @pl.when(jj < NJ - 1)
            def _():
                pltpu.async_copy(
                    buf, out_hbm.at[pl.ds(row0, 8), pl.ds(col0, CW)], semo)

            @pl.when(jj == NJ - 1)
            def _():
                pltpu.async_copy(
                    buf.at[:, pl.ds(0, CWLAST)],
                    out_hbm.at[pl.ds(row0, 8), pl.ds(col0, CWLAST)], semo)

    set0 = (ibuf0, sbuf0, mrow0)
    set1 = (ibuf1, sbuf1, mrow1)

    def chunk_loop(k2, carry):
        sub_step(2 * k2, cb0, semx0, semo0, set0, set1)
        sub_step(2 * k2 + 1, cb1, semx1, semo1, set1, set0)
        return carry

    lax.fori_loop(0, NBP // NW // 2, chunk_loop, i32(0))

    lastk = lax.div(i32(NB - 1) - wid, i32(NW))
    lastc = lastk * NW + wid
    parity = jnp.bitwise_and(lastk, i32(1))

    @pl.when(parity == 0)
    def _():
        drain_out(cb0, semo0, lastc)
        drain_out(cb1, semo1, lastc - NW)

    @pl.when(parity == 1)
    def _():
        drain_out(cb1, semo1, lastc)
        drain_out(cb0, semo0, lastc - NW)


def kernel(x, index, source):
    i32 = jnp.int32
    f32 = jnp.float32
    xt = x.T
    idx = index.astype(i32)
    mesh = plsc.VectorSubcoreMesh(core_axis_name="c", subcore_axis_name="s")
    params_a = pltpu.CompilerParams(needs_layout_passes=False)
    params_b = pltpu.CompilerParams(needs_layout_passes=False,
                                    use_tc_tiling_on_sc=True)

    phase_a = functools.partial(
        pl.kernel, mesh=mesh, compiler_params=params_a,
        out_type=[jax.ShapeDtypeStruct((PART_LEN,), i32),
                  jax.ShapeDtypeStruct((PART_LEN,), f32),
                  jax.ShapeDtypeStruct((META_LEN,), i32)],
        scratch_types=[
            pltpu.VMEM((WIN,), i32),
            pltpu.VMEM((WIN,), f32),
            pltpu.VMEM((CAP,), i32),
            pltpu.VMEM((CAP,), f32),
            pltpu.VMEM((NBP,), i32),
            pltpu.VMEM((NBP,), i32),
            pltpu.VMEM((NBP,), i32),
            pltpu.VMEM((NBP // SEG, SEG), i32),
            pltpu.SemaphoreType.DMA,
        ])(_phase_a)
    ipart, spart, meta = phase_a(idx, source)

    phase_b = functools.partial(
        pl.kernel, mesh=mesh, compiler_params=params_b,
        out_type=jax.ShapeDtypeStruct((NCOL, NROW), f32),
        scratch_types=[
            pltpu.VMEM((8, CW), f32),
            pltpu.VMEM((8, CW), f32),
            pltpu.VMEM((NW, SEG), i32),
            pltpu.VMEM((NW, SEG), f32),
            pltpu.VMEM((NW,), i32),
            pltpu.VMEM((NW, SEG), i32),
            pltpu.VMEM((NW, SEG), f32),
            pltpu.VMEM((NW,), i32),
            pltpu.SemaphoreType.DMA,
            pltpu.SemaphoreType.DMA,
            pltpu.SemaphoreType.DMA,
            pltpu.SemaphoreType.DMA,
            pltpu.SemaphoreType.DMA,
        ])(_phase_b)
    out = phase_b(xt, ipart, spart, meta)
    return out.T
